# Initial kernel scaffold; baseline (speedup 1.0000x reference)
import jax
import jax.numpy as jnp
from jax.experimental import pallas as pl


def kernel(x, edge_index, enc_W0, enc_b0, enc_W1, enc_b1, conv_W0, conv_b0, conv_W1, conv_b1, conv_W2, conv_b2, pred_W0, pred_b0, pred_W1, pred_b1):
    raise NotImplementedError("write your pallas kernel here")


# speedup vs baseline: 1.0596x; 1.0596x over previous
import functools

import jax
import jax.numpy as jnp
from jax.experimental import pallas as pl
from jax.experimental.pallas import tpu as pltpu

N = 10000
E = 320000
D = 128
H = 128


def _encoder_body(x_ref, w0_ref, b0_ref, w1_ref, b1_ref, out_ref):
    h = jnp.maximum(
        jnp.dot(x_ref[...], w0_ref[...], preferred_element_type=jnp.float32)
        + b0_ref[...], 0.0)
    out_ref[...] = jnp.maximum(
        jnp.dot(h, w1_ref[...], preferred_element_type=jnp.float32)
        + b1_ref[...], 0.0)


def _encoder(x, w0, b0, w1, b1):
    return pl.pallas_call(
        _encoder_body,
        out_shape=jax.ShapeDtypeStruct((N, H), jnp.float32),
    )(x, w0, b0.reshape(1, H), w1, b1.reshape(1, H))


def _conv_body(m_ref, dinv_ref, h_ref, w_ref, b_ref, out_ref):
    m = m_ref[...] * dinv_ref[...]
    out_ref[...] = jnp.maximum(
        jnp.dot(m, w_ref[...], preferred_element_type=jnp.float32)
        + b_ref[...] + h_ref[...], 0.0)


def _conv(m, dinv, h, w, b):
    return pl.pallas_call(
        _conv_body,
        out_shape=jax.ShapeDtypeStruct((N, H), jnp.float32),
    )(m, dinv, h, w, b.reshape(1, H))


def _conv_pred_body(m_ref, dinv_ref, h_ref, w_ref, b_ref,
                    pw0_ref, pb0_ref, pw1_ref, pb1_ref, out_ref):
    m = m_ref[...] * dinv_ref[...]
    hn = jnp.maximum(
        jnp.dot(m, w_ref[...], preferred_element_type=jnp.float32)
        + b_ref[...] + h_ref[...], 0.0)
    obj = jnp.mean(hn, axis=0, keepdims=True)
    z = jnp.maximum(
        jnp.dot(obj, pw0_ref[...], preferred_element_type=jnp.float32)
        + pb0_ref[...], 0.0)
    out_ref[...] = (
        jnp.dot(z, pw1_ref[...], preferred_element_type=jnp.float32)
        + pb1_ref[...])


def _conv_pred(m, dinv, h, w, b, pw0, pb0, pw1, pb1):
    out = pl.pallas_call(
        _conv_pred_body,
        out_shape=jax.ShapeDtypeStruct((1, 1), jnp.float32),
    )(m, dinv, h, w, b.reshape(1, H),
      pw0, pb0.reshape(1, H), pw1, pb1.reshape(1, 1))
    return out.reshape(())


def _segment_sums(h, src, dst):
    m = jax.ops.segment_sum(h[src], dst, num_segments=N)
    return m


def kernel(x, edge_index, enc_W0, enc_b0, enc_W1, enc_b1,
           conv_W0, conv_b0, conv_W1, conv_b1, conv_W2, conv_b2,
           pred_W0, pred_b0, pred_W1, pred_b1):
    src = edge_index[0]
    dst = edge_index[1]

    h = _encoder(x, enc_W0, enc_b0, enc_W1, enc_b1)

    deg = jax.ops.segment_sum(jnp.ones((E,), jnp.float32), dst, num_segments=N)
    dinv = (1.0 / jnp.clip(deg, 1.0, None))[:, None]

    m = _segment_sums(h, src, dst)
    h = _conv(m, dinv, h, conv_W0, conv_b0)
    m = _segment_sums(h, src, dst)
    h = _conv(m, dinv, h, conv_W1, conv_b1)
    m = _segment_sums(h, src, dst)
    return _conv_pred(m, dinv, h, conv_W2, conv_b2,
                      pred_W0, pred_b0, pred_W1, pred_b1)


# speedup vs baseline: 3.3761x; 3.1863x over previous
import functools

import jax
import jax.numpy as jnp
from jax import lax
from jax.experimental import pallas as pl
from jax.experimental.pallas import tpu as pltpu
from jax.experimental.pallas import tpu_sc as plsc

N = 10000
E = 320000
D = 128
H = 128

NC = 2
NS = 16
NW = NC * NS
L = 16

C = 128
CH = 80
EPT = C * CH
EPAD = NW * EPT
NP = 10240
RPT = NP // NS
DW = 16

_MESH = plsc.VectorSubcoreMesh(core_axis_name="c", subcore_axis_name="s")


def _msg_body(h_hbm, ei_hbm, m_out,
              pk_idx, src_cur, dst_cur, rows, m_acc, sem0, sem1):
    sems = (sem0, sem1)
    c = lax.axis_index("c")
    s = lax.axis_index("s")
    wid = c * NS + s

    pltpu.sync_copy(ei_hbm.at[wid], pk_idx)

    z16 = jnp.zeros((L,), jnp.float32)

    def _zrow(i, carry):
        for jj in range(H // L):
            rows[0, i, pl.ds(jj * L, L)] = z16
        return carry
    lax.fori_loop(0, C, _zrow, 0)
    for k in range(RPT // C):
        pltpu.sync_copy(rows.at[0], m_acc.at[pl.ds(s * RPT + k * C, C)])

    plsc.subcore_barrier()

    def _unpack(j, b):
        for t in range(C // L):
            p = pk_idx[j, pl.ds(t * L, L)]
            src_cur[b, pl.ds(t * L, L)] = p & 0x3FFF
            dst_cur[b, pl.ds(t * L, L)] = lax.shift_right_logical(p, 14)

    def _gather(j, b):
        _unpack(j, b)
        return pltpu.async_copy(h_hbm.at[src_cur.at[b]], rows.at[b], sems[b])

    _gather(0, 0)
    _gather(1, 1)

    def _step(j2, carry):
        for b in range(2):
            j = j2 * 2 + b
            pltpu.make_async_copy(h_hbm.at[src_cur.at[b]], rows.at[b],
                                  sems[b]).wait()
            pltpu.sync_copy(rows.at[b], m_acc.at[dst_cur.at[b]], add=True)
            jn = j + 2

            @pl.when(jn < CH)
            def _():
                _gather(jn, b)
        return carry
    lax.fori_loop(0, CH // 2, _step, 0)

    plsc.subcore_barrier()

    pltpu.sync_copy(m_acc.at[pl.ds(s * RPT, RPT)],
                    m_out.at[c, pl.ds(s * RPT, RPT)])


_msg = pl.kernel(
    _msg_body,
    out_type=jax.ShapeDtypeStruct((NC, NP, H), jnp.float32),
    mesh=_MESH,
    scratch_types=[
        pltpu.VMEM((CH, C), jnp.int32),
        pltpu.VMEM((2, C), jnp.int32),
        pltpu.VMEM((2, C), jnp.int32),
        pltpu.VMEM((2, C, H), jnp.float32),
        pltpu.VMEM_SHARED((NP, H), jnp.float32),
        pltpu.SemaphoreType.DMA,
        pltpu.SemaphoreType.DMA,
    ],
)


def _deg_body(ei_hbm, deg_out, dst_idx, ones_b, zero_b, deg_acc):
    c = lax.axis_index("c")
    s = lax.axis_index("s")
    wid = c * NS + s

    pltpu.sync_copy(ei_hbm.at[wid], dst_idx)

    def _unpack(i, carry):
        j = i // (C // L)
        t = (i % (C // L)) * L
        dst_idx[j, pl.ds(t, L)] = lax.shift_right_logical(
            dst_idx[j, pl.ds(t, L)], 14)
        return carry
    lax.fori_loop(0, CH * (C // L), _unpack, 0)

    one16 = jnp.ones((L,), jnp.float32)
    z16 = jnp.zeros((L,), jnp.float32)

    def _fill(i, carry):
        for jj in range(H // L):
            ones_b[i, pl.ds(jj * L, L)] = one16
            zero_b[i, pl.ds(jj * L, L)] = z16
        return carry
    lax.fori_loop(0, C, _fill, 0)
    for k in range(RPT // C):
        pltpu.sync_copy(zero_b, deg_acc.at[pl.ds(s * RPT + k * C, C)])

    plsc.subcore_barrier()

    def _step(j, carry):
        pltpu.sync_copy(ones_b, deg_acc.at[dst_idx.at[j]], add=True)
        return carry
    lax.fori_loop(0, CH, _step, 0)

    plsc.subcore_barrier()

    pltpu.sync_copy(deg_acc.at[pl.ds(s * RPT, RPT)],
                    deg_out.at[c, pl.ds(s * RPT, RPT)])


_deg = pl.kernel(
    _deg_body,
    out_type=jax.ShapeDtypeStruct((NC, NP, H), jnp.float32),
    mesh=_MESH,
    scratch_types=[
        pltpu.VMEM((CH, C), jnp.int32),
        pltpu.VMEM((C, H), jnp.float32),
        pltpu.VMEM((C, H), jnp.float32),
        pltpu.VMEM_SHARED((NP, H), jnp.float32),
    ],
)


def _encoder_body(x_ref, w0_ref, b0_ref, w1_ref, b1_ref, out_ref):
    h = jnp.maximum(
        jnp.dot(x_ref[...], w0_ref[...], preferred_element_type=jnp.float32)
        + b0_ref[...], 0.0)
    out_ref[...] = jnp.maximum(
        jnp.dot(h, w1_ref[...], preferred_element_type=jnp.float32)
        + b1_ref[...], 0.0)


def _encoder(x, w0, b0, w1, b1):
    return pl.pallas_call(
        _encoder_body,
        out_shape=jax.ShapeDtypeStruct((N, H), jnp.float32),
    )(x, w0, b0.reshape(1, H), w1, b1.reshape(1, H))


def _mean_msg(m_ref, degp_ref):
    dp = degp_ref[...]
    deg = dp[0, :N, 0] + dp[1, :N, 0]
    dinv = 1.0 / jnp.maximum(deg, 1.0)
    return (m_ref[0, :N, :] + m_ref[1, :N, :]) * dinv[:, None]


def _conv_body(m_ref, degp_ref, h_ref, w_ref, b_ref, out_ref):
    m = _mean_msg(m_ref, degp_ref)
    out_ref[...] = jnp.maximum(
        jnp.dot(m, w_ref[...], preferred_element_type=jnp.float32)
        + b_ref[...] + h_ref[...], 0.0)


def _conv(m, degp, h, w, b):
    return pl.pallas_call(
        _conv_body,
        out_shape=jax.ShapeDtypeStruct((N, H), jnp.float32),
    )(m, degp, h, w, b.reshape(1, H))


def _conv_pred_body(m_ref, degp_ref, h_ref, w_ref, b_ref,
                    pw0_ref, pb0_ref, pw1_ref, pb1_ref, out_ref):
    m = _mean_msg(m_ref, degp_ref)
    hn = jnp.maximum(
        jnp.dot(m, w_ref[...], preferred_element_type=jnp.float32)
        + b_ref[...] + h_ref[...], 0.0)
    obj = jnp.mean(hn, axis=0, keepdims=True)
    z = jnp.maximum(
        jnp.dot(obj, pw0_ref[...], preferred_element_type=jnp.float32)
        + pb0_ref[...], 0.0)
    out_ref[...] = (
        jnp.dot(z, pw1_ref[...], preferred_element_type=jnp.float32)
        + pb1_ref[...])


def _conv_pred(m, degp, h, w, b, pw0, pb0, pw1, pb1):
    out = pl.pallas_call(
        _conv_pred_body,
        out_shape=jax.ShapeDtypeStruct((1, 1), jnp.float32),
    )(m, degp, h, w, b.reshape(1, H),
      pw0, pb0.reshape(1, H), pw1, pb1.reshape(1, 1))
    return out.reshape(())


def kernel(x, edge_index, enc_W0, enc_b0, enc_W1, enc_b1,
           conv_W0, conv_b0, conv_W1, conv_b1, conv_W2, conv_b2,
           pred_W0, pred_b0, pred_W1, pred_b1):
    src = edge_index[0]
    dst = edge_index[1]
    pad = EPAD - E
    src_p = jnp.concatenate([src, jnp.zeros((pad,), jnp.int32)])
    dst_p = jnp.concatenate(
        [dst, N + (jnp.arange(pad, dtype=jnp.int32) % (NP - N))])
    ei_p = (src_p | (dst_p << 14)).reshape(NW, CH, C)

    h = _encoder(x, enc_W0, enc_b0, enc_W1, enc_b1)
    degp = _deg(ei_p)

    m = _msg(h, ei_p)
    h = _conv(m, degp, h, conv_W0, conv_b0)
    m = _msg(h, ei_p)
    h = _conv(m, degp, h, conv_W1, conv_b1)
    m = _msg(h, ei_p)
    return _conv_pred(m, degp, h, conv_W2, conv_b2,
                      pred_W0, pred_b0, pred_W1, pred_b1)


# speedup vs baseline: 3.5398x; 1.0485x over previous
import functools

import jax
import jax.numpy as jnp
from jax import lax
from jax.experimental import pallas as pl
from jax.experimental.pallas import tpu as pltpu
from jax.experimental.pallas import tpu_sc as plsc

N = 10000
E = 320000
D = 128
H = 128

NC = 2
NS = 16
NW = NC * NS
L = 16

C = 128
CH0 = 120
CH1 = 40
TCH = CH0 + CH1
NCH = NS * TCH
EPAD = NCH * C
DCH = NCH // NW
NP = 10240
RPT = NP // NS

_MESH = plsc.VectorSubcoreMesh(core_axis_name="c", subcore_axis_name="s")


def _msg_body(h_hbm, ei_hbm, m_out,
              pk_idx, src_cur, dst_cur, rows, m_acc, sem0, sem1):
    sems = (sem0, sem1)
    c = lax.axis_index("c")
    s = lax.axis_index("s")
    nch = jnp.where(c == 0, CH0, CH1)

    @pl.when(c == 0)
    def _():
        pltpu.sync_copy(ei_hbm.at[pl.ds(s * CH0, CH0)], pk_idx)

    @pl.when(c == 1)
    def _():
        pltpu.sync_copy(ei_hbm.at[pl.ds(NS * CH0 + s * CH1, CH1)],
                        pk_idx.at[pl.ds(0, CH1)])

    z16 = jnp.zeros((L,), jnp.float32)

    def _zrow(i, carry):
        for jj in range(H // L):
            rows[0, i, pl.ds(jj * L, L)] = z16
        return carry
    lax.fori_loop(0, C, _zrow, 0)
    for k in range(RPT // C):
        pltpu.sync_copy(rows.at[0], m_acc.at[pl.ds(s * RPT + k * C, C)])

    plsc.subcore_barrier()

    def _unpack(j, b):
        for t in range(C // L):
            p = pk_idx[j, pl.ds(t * L, L)]
            src_cur[b, pl.ds(t * L, L)] = p & 0x3FFF
            dst_cur[b, pl.ds(t * L, L)] = lax.shift_right_logical(p, 14)

    def _gather(j, b):
        _unpack(j, b)
        return pltpu.async_copy(h_hbm.at[src_cur.at[b]], rows.at[b], sems[b])

    _gather(0, 0)
    _gather(1, 1)

    def _step(j2, carry):
        for b in range(2):
            j = j2 * 2 + b
            pltpu.make_async_copy(h_hbm.at[src_cur.at[b]], rows.at[b],
                                  sems[b]).wait()
            pltpu.sync_copy(rows.at[b], m_acc.at[dst_cur.at[b]], add=True)
            jn = j + 2

            @pl.when(jn < nch)
            def _():
                _gather(jn, b)
        return carry
    lax.fori_loop(0, nch // 2, _step, 0)

    plsc.subcore_barrier()

    pltpu.sync_copy(m_acc.at[pl.ds(s * RPT, RPT)],
                    m_out.at[c, pl.ds(s * RPT, RPT)])


_msg = pl.kernel(
    _msg_body,
    out_type=jax.ShapeDtypeStruct((NC, NP, H), jnp.float32),
    mesh=_MESH,
    scratch_types=[
        pltpu.VMEM((CH0, C), jnp.int32),
        pltpu.VMEM((2, C), jnp.int32),
        pltpu.VMEM((2, C), jnp.int32),
        pltpu.VMEM((2, C, H), jnp.float32),
        pltpu.VMEM_SHARED((NP, H), jnp.float32),
        pltpu.SemaphoreType.DMA,
        pltpu.SemaphoreType.DMA,
    ],
)


def _deg_body(ei_hbm, deg_out, dst_idx, ones_b, zero_b, deg_acc):
    c = lax.axis_index("c")
    s = lax.axis_index("s")
    wid = c * NS + s

    pltpu.sync_copy(ei_hbm.at[pl.ds(wid * DCH, DCH)], dst_idx)

    def _unpack(i, carry):
        j = i // (C // L)
        t = (i % (C // L)) * L
        dst_idx[j, pl.ds(t, L)] = lax.shift_right_logical(
            dst_idx[j, pl.ds(t, L)], 14)
        return carry
    lax.fori_loop(0, DCH * (C // L), _unpack, 0)

    one16 = jnp.ones((L,), jnp.float32)
    z16 = jnp.zeros((L,), jnp.float32)

    def _fill(i, carry):
        for jj in range(H // L):
            ones_b[i, pl.ds(jj * L, L)] = one16
            zero_b[i, pl.ds(jj * L, L)] = z16
        return carry
    lax.fori_loop(0, C, _fill, 0)
    for k in range(RPT // C):
        pltpu.sync_copy(zero_b, deg_acc.at[pl.ds(s * RPT + k * C, C)])

    plsc.subcore_barrier()

    def _step(j, carry):
        pltpu.sync_copy(ones_b, deg_acc.at[dst_idx.at[j]], add=True)
        return carry
    lax.fori_loop(0, DCH, _step, 0)

    plsc.subcore_barrier()

    pltpu.sync_copy(deg_acc.at[pl.ds(s * RPT, RPT)],
                    deg_out.at[c, pl.ds(s * RPT, RPT)])


_deg = pl.kernel(
    _deg_body,
    out_type=jax.ShapeDtypeStruct((NC, NP, H), jnp.float32),
    mesh=_MESH,
    scratch_types=[
        pltpu.VMEM((DCH, C), jnp.int32),
        pltpu.VMEM((C, H), jnp.float32),
        pltpu.VMEM((C, H), jnp.float32),
        pltpu.VMEM_SHARED((NP, H), jnp.float32),
    ],
)


def _encoder_body(x_ref, w0_ref, b0_ref, w1_ref, b1_ref, out_ref):
    h = jnp.maximum(
        jnp.dot(x_ref[...], w0_ref[...], preferred_element_type=jnp.float32)
        + b0_ref[...], 0.0)
    out_ref[...] = jnp.maximum(
        jnp.dot(h, w1_ref[...], preferred_element_type=jnp.float32)
        + b1_ref[...], 0.0)


def _encoder(x, w0, b0, w1, b1):
    return pl.pallas_call(
        _encoder_body,
        out_shape=jax.ShapeDtypeStruct((N, H), jnp.float32),
    )(x, w0, b0.reshape(1, H), w1, b1.reshape(1, H))


def _mean_msg(m_ref, degp_ref):
    dp = degp_ref[...]
    deg = dp[0, :N, 0] + dp[1, :N, 0]
    dinv = 1.0 / jnp.maximum(deg, 1.0)
    return (m_ref[0, :N, :] + m_ref[1, :N, :]) * dinv[:, None]


def _conv_body(m_ref, degp_ref, h_ref, w_ref, b_ref, out_ref):
    m = _mean_msg(m_ref, degp_ref)
    out_ref[...] = jnp.maximum(
        jnp.dot(m, w_ref[...], preferred_element_type=jnp.float32)
        + b_ref[...] + h_ref[...], 0.0)


def _conv(m, degp, h, w, b):
    return pl.pallas_call(
        _conv_body,
        out_shape=jax.ShapeDtypeStruct((N, H), jnp.float32),
    )(m, degp, h, w, b.reshape(1, H))


def _conv_pred_body(m_ref, degp_ref, h_ref, w_ref, b_ref,
                    pw0_ref, pb0_ref, pw1_ref, pb1_ref, out_ref):
    m = _mean_msg(m_ref, degp_ref)
    hn = jnp.maximum(
        jnp.dot(m, w_ref[...], preferred_element_type=jnp.float32)
        + b_ref[...] + h_ref[...], 0.0)
    obj = jnp.mean(hn, axis=0, keepdims=True)
    z = jnp.maximum(
        jnp.dot(obj, pw0_ref[...], preferred_element_type=jnp.float32)
        + pb0_ref[...], 0.0)
    out_ref[...] = (
        jnp.dot(z, pw1_ref[...], preferred_element_type=jnp.float32)
        + pb1_ref[...])


def _conv_pred(m, degp, h, w, b, pw0, pb0, pw1, pb1):
    out = pl.pallas_call(
        _conv_pred_body,
        out_shape=jax.ShapeDtypeStruct((1, 1), jnp.float32),
    )(m, degp, h, w, b.reshape(1, H),
      pw0, pb0.reshape(1, H), pw1, pb1.reshape(1, 1))
    return out.reshape(())


def kernel(x, edge_index, enc_W0, enc_b0, enc_W1, enc_b1,
           conv_W0, conv_b0, conv_W1, conv_b1, conv_W2, conv_b2,
           pred_W0, pred_b0, pred_W1, pred_b1):
    src = edge_index[0]
    dst = edge_index[1]
    pad = EPAD - E
    src_p = jnp.concatenate([src, jnp.zeros((pad,), jnp.int32)])
    dst_p = jnp.concatenate(
        [dst, N + (jnp.arange(pad, dtype=jnp.int32) % (NP - N))])
    ei_p = (src_p | (dst_p << 14)).reshape(NCH, C)

    h = _encoder(x, enc_W0, enc_b0, enc_W1, enc_b1)
    degp = _deg(ei_p)

    m = _msg(h, ei_p)
    h = _conv(m, degp, h, conv_W0, conv_b0)
    m = _msg(h, ei_p)
    h = _conv(m, degp, h, conv_W1, conv_b1)
    m = _msg(h, ei_p)
    return _conv_pred(m, degp, h, conv_W2, conv_b2,
                      pred_W0, pred_b0, pred_W1, pred_b1)


# speedup vs baseline: 3.5403x; 1.0001x over previous
import functools

import jax
import jax.numpy as jnp
from jax import lax
from jax.experimental import pallas as pl
from jax.experimental.pallas import tpu as pltpu
from jax.experimental.pallas import tpu_sc as plsc

N = 10000
E = 320000
D = 128
H = 128

NC = 2
NS = 16
NW = NC * NS
L = 16

C = 128
CH0 = 120
CH1 = 40
TCH = CH0 + CH1
NCH = NS * TCH
EPAD = NCH * C
DCH = NCH // NW
NP = 10240
RPT = NP // NS

_MESH = plsc.VectorSubcoreMesh(core_axis_name="c", subcore_axis_name="s")


def _msg_body(h_hbm, ei_hbm, m_out,
              pk_idx, src_cur, dst_cur, rows, m_acc, sem0, sem1):
    sems = (sem0, sem1)
    c = lax.axis_index("c")
    s = lax.axis_index("s")
    nch = jnp.where(c == 0, CH0, CH1)

    @pl.when(c == 0)
    def _():
        pltpu.sync_copy(ei_hbm.at[pl.ds(s * CH0, CH0)], pk_idx)

    @pl.when(c == 1)
    def _():
        pltpu.sync_copy(ei_hbm.at[pl.ds(NS * CH0 + s * CH1, CH1)],
                        pk_idx.at[pl.ds(0, CH1)])

    z16 = jnp.zeros((L,), jnp.float32)

    def _zrow(i, carry):
        for jj in range(H // L):
            rows[0, i, pl.ds(jj * L, L)] = z16
        return carry
    lax.fori_loop(0, C, _zrow, 0)
    for k in range(RPT // C):
        pltpu.sync_copy(rows.at[0], m_acc.at[pl.ds(s * RPT + k * C, C)])

    plsc.subcore_barrier()

    def _unpack(j, b):
        for t in range(C // L):
            p = pk_idx[j, pl.ds(t * L, L)]
            src_cur[b, pl.ds(t * L, L)] = p & 0x3FFF
            dst_cur[b, pl.ds(t * L, L)] = lax.shift_right_logical(p, 14)

    def _gather(j, b):
        _unpack(j, b)
        return pltpu.async_copy(h_hbm.at[src_cur.at[b]], rows.at[b], sems[b])

    def _step(j2, carry):
        for b in range(2):
            j = j2 * 2 + b
            pltpu.make_async_copy(h_hbm.at[src_cur.at[b]], rows.at[b],
                                  sems[b]).wait()
            pltpu.sync_copy(rows.at[b], m_acc.at[dst_cur.at[b]], add=True)
            jn = j + 2

            @pl.when(jn < nch)
            def _():
                _gather(jn, b)
        return carry

    with jax.named_scope("msg_loop"):
        _gather(0, 0)
        _gather(1, 1)
        lax.fori_loop(0, nch // 2, _step, 0)

    with jax.named_scope("msg_bar"):
        plsc.subcore_barrier()

    with jax.named_scope("msg_out"):
        pltpu.sync_copy(m_acc.at[pl.ds(s * RPT, RPT)],
                        m_out.at[c, pl.ds(s * RPT, RPT)])


_msg = pl.kernel(
    _msg_body,
    out_type=jax.ShapeDtypeStruct((NC, NP, H), jnp.float32),
    mesh=_MESH,
    scratch_types=[
        pltpu.VMEM((CH0, C), jnp.int32),
        pltpu.VMEM((2, C), jnp.int32),
        pltpu.VMEM((2, C), jnp.int32),
        pltpu.VMEM((2, C, H), jnp.float32),
        pltpu.VMEM_SHARED((NP, H), jnp.float32),
        pltpu.SemaphoreType.DMA,
        pltpu.SemaphoreType.DMA,
    ],
)


def _deg_body(ei_hbm, deg_out, dst_idx, ones_b, zero_b, deg_acc):
    c = lax.axis_index("c")
    s = lax.axis_index("s")
    wid = c * NS + s

    pltpu.sync_copy(ei_hbm.at[pl.ds(wid * DCH, DCH)], dst_idx)

    def _unpack(i, carry):
        j = i // (C // L)
        t = (i % (C // L)) * L
        dst_idx[j, pl.ds(t, L)] = lax.shift_right_logical(
            dst_idx[j, pl.ds(t, L)], 14)
        return carry
    lax.fori_loop(0, DCH * (C // L), _unpack, 0)

    one16 = jnp.ones((L,), jnp.float32)
    z16 = jnp.zeros((L,), jnp.float32)

    def _fill(i, carry):
        for jj in range(H // L):
            ones_b[i, pl.ds(jj * L, L)] = one16
            zero_b[i, pl.ds(jj * L, L)] = z16
        return carry
    lax.fori_loop(0, C, _fill, 0)
    for k in range(RPT // C):
        pltpu.sync_copy(zero_b, deg_acc.at[pl.ds(s * RPT + k * C, C)])

    plsc.subcore_barrier()

    def _step(j, carry):
        pltpu.sync_copy(ones_b, deg_acc.at[dst_idx.at[j]], add=True)
        return carry
    lax.fori_loop(0, DCH, _step, 0)

    plsc.subcore_barrier()

    pltpu.sync_copy(deg_acc.at[pl.ds(s * RPT, RPT)],
                    deg_out.at[c, pl.ds(s * RPT, RPT)])


_deg = pl.kernel(
    _deg_body,
    out_type=jax.ShapeDtypeStruct((NC, NP, H), jnp.float32),
    mesh=_MESH,
    scratch_types=[
        pltpu.VMEM((DCH, C), jnp.int32),
        pltpu.VMEM((C, H), jnp.float32),
        pltpu.VMEM((C, H), jnp.float32),
        pltpu.VMEM_SHARED((NP, H), jnp.float32),
    ],
)


def _encoder_body(x_ref, w0_ref, b0_ref, w1_ref, b1_ref, out_ref):
    h = jnp.maximum(
        jnp.dot(x_ref[...], w0_ref[...], preferred_element_type=jnp.float32)
        + b0_ref[...], 0.0)
    out_ref[...] = jnp.maximum(
        jnp.dot(h, w1_ref[...], preferred_element_type=jnp.float32)
        + b1_ref[...], 0.0)


def _encoder(x, w0, b0, w1, b1):
    return pl.pallas_call(
        _encoder_body,
        out_shape=jax.ShapeDtypeStruct((N, H), jnp.float32),
    )(x, w0, b0.reshape(1, H), w1, b1.reshape(1, H))


def _mean_msg(m_ref, degp_ref):
    dp = degp_ref[...]
    deg = dp[0, :N, 0] + dp[1, :N, 0]
    dinv = 1.0 / jnp.maximum(deg, 1.0)
    return (m_ref[0, :N, :] + m_ref[1, :N, :]) * dinv[:, None]


def _conv_body(m_ref, degp_ref, h_ref, w_ref, b_ref, out_ref):
    m = _mean_msg(m_ref, degp_ref)
    out_ref[...] = jnp.maximum(
        jnp.dot(m, w_ref[...], preferred_element_type=jnp.float32)
        + b_ref[...] + h_ref[...], 0.0)


def _conv(m, degp, h, w, b):
    return pl.pallas_call(
        _conv_body,
        out_shape=jax.ShapeDtypeStruct((N, H), jnp.float32),
    )(m, degp, h, w, b.reshape(1, H))


def _conv_pred_body(m_ref, degp_ref, h_ref, w_ref, b_ref,
                    pw0_ref, pb0_ref, pw1_ref, pb1_ref, out_ref):
    m = _mean_msg(m_ref, degp_ref)
    hn = jnp.maximum(
        jnp.dot(m, w_ref[...], preferred_element_type=jnp.float32)
        + b_ref[...] + h_ref[...], 0.0)
    obj = jnp.mean(hn, axis=0, keepdims=True)
    z = jnp.maximum(
        jnp.dot(obj, pw0_ref[...], preferred_element_type=jnp.float32)
        + pb0_ref[...], 0.0)
    out_ref[...] = (
        jnp.dot(z, pw1_ref[...], preferred_element_type=jnp.float32)
        + pb1_ref[...])


def _conv_pred(m, degp, h, w, b, pw0, pb0, pw1, pb1):
    out = pl.pallas_call(
        _conv_pred_body,
        out_shape=jax.ShapeDtypeStruct((1, 1), jnp.float32),
    )(m, degp, h, w, b.reshape(1, H),
      pw0, pb0.reshape(1, H), pw1, pb1.reshape(1, 1))
    return out.reshape(())


def kernel(x, edge_index, enc_W0, enc_b0, enc_W1, enc_b1,
           conv_W0, conv_b0, conv_W1, conv_b1, conv_W2, conv_b2,
           pred_W0, pred_b0, pred_W1, pred_b1):
    src = edge_index[0]
    dst = edge_index[1]
    pad = EPAD - E
    src_p = jnp.concatenate([src, jnp.zeros((pad,), jnp.int32)])
    dst_p = jnp.concatenate(
        [dst, N + (jnp.arange(pad, dtype=jnp.int32) % (NP - N))])
    ei_p = (src_p | (dst_p << 14)).reshape(NCH, C)

    h = _encoder(x, enc_W0, enc_b0, enc_W1, enc_b1)
    degp = _deg(ei_p)

    m = _msg(h, ei_p)
    h = _conv(m, degp, h, conv_W0, conv_b0)
    m = _msg(h, ei_p)
    h = _conv(m, degp, h, conv_W1, conv_b1)
    m = _msg(h, ei_p)
    return _conv_pred(m, degp, h, conv_W2, conv_b2,
                      pred_W0, pred_b0, pred_W1, pred_b1)


# speedup vs baseline: 10.7872x; 3.0470x over previous
import functools

import jax
import jax.numpy as jnp
from jax import lax
from jax.experimental import pallas as pl
from jax.experimental.pallas import tpu as pltpu
from jax.experimental.pallas import tpu_sc as plsc

N = 10000
E = 320000
D = 128
H = 128

NC = 2
NS = 16
NW = NC * NS
L = 16

C = 128
CH0 = 80
CH1 = 80
TCH = CH0 + CH1
NCH = NS * TCH
EPAD = NCH * C
DCH = NCH // NW
NP = 10240
RPT = NP // NS

_MESH = plsc.VectorSubcoreMesh(core_axis_name="c", subcore_axis_name="s")


def _msg_body(h_hbm, ei_hbm, m_out,
              pk_idx, src_cur, dst_cur, rows, m_acc, sem0, sem1):
    sems = (sem0, sem1)
    c = lax.axis_index("c")
    s = lax.axis_index("s")
    nch = jnp.where(c == 0, CH0, CH1)

    @pl.when(c == 0)
    def _():
        pltpu.sync_copy(ei_hbm.at[pl.ds(s * CH0, CH0)], pk_idx)

    @pl.when(c == 1)
    def _():
        pltpu.sync_copy(ei_hbm.at[pl.ds(NS * CH0 + s * CH1, CH1)],
                        pk_idx.at[pl.ds(0, CH1)])

    z16 = jnp.zeros((L,), jnp.float32)

    def _zrow(i, carry):
        for jj in range(H // L):
            rows[0, i, pl.ds(jj * L, L)] = z16
        return carry
    lax.fori_loop(0, C, _zrow, 0)
    for k in range(RPT // C):
        pltpu.sync_copy(rows.at[0], m_acc.at[pl.ds(s * RPT + k * C, C)])

    plsc.subcore_barrier()

    def _unpack(j, b):
        for t in range(C // L):
            p = pk_idx[j, pl.ds(t * L, L)]
            src_cur[b, pl.ds(t * L, L)] = p & 0x3FFF
            dst_cur[b, pl.ds(t * L, L)] = lax.shift_right_logical(p, 14)

    def _gather(j, b):
        _unpack(j, b)
        return pltpu.async_copy(h_hbm.at[src_cur.at[b]], rows.at[b], sems[b])

    def _step(j2, carry):
        for b in range(2):
            j = j2 * 2 + b
            pltpu.make_async_copy(h_hbm.at[src_cur.at[b]], rows.at[b],
                                  sems[b]).wait()
            pltpu.sync_copy(rows.at[b], m_acc.at[dst_cur.at[b]], add=True)
            jn = j + 2

            @pl.when(jn < nch)
            def _():
                _gather(jn, b)
        return carry

    with jax.named_scope("msg_loop"):
        _gather(0, 0)
        _gather(1, 1)
        lax.fori_loop(0, nch // 2, _step, 0)

    with jax.named_scope("msg_bar"):
        plsc.subcore_barrier()

    with jax.named_scope("msg_out"):
        pltpu.sync_copy(m_acc.at[pl.ds(s * RPT, RPT)],
                        m_out.at[c, pl.ds(s * RPT, RPT)])


_msg = pl.kernel(
    _msg_body,
    out_type=jax.ShapeDtypeStruct((NC, NP, H), jnp.float32),
    mesh=_MESH,
    scratch_types=[
        pltpu.VMEM((CH0, C), jnp.int32),
        pltpu.VMEM((2, C), jnp.int32),
        pltpu.VMEM((2, C), jnp.int32),
        pltpu.VMEM((2, C, H), jnp.float32),
        pltpu.VMEM_SHARED((NP, H), jnp.float32),
        pltpu.SemaphoreType.DMA,
        pltpu.SemaphoreType.DMA,
    ],
)


def _deg_body(ei_hbm, deg_out, dst_idx, ones_b, zero_b, deg_acc):
    c = lax.axis_index("c")
    s = lax.axis_index("s")
    wid = c * NS + s

    pltpu.sync_copy(ei_hbm.at[pl.ds(wid * DCH, DCH)], dst_idx)

    def _unpack(i, carry):
        j = i // (C // L)
        t = (i % (C // L)) * L
        dst_idx[j, pl.ds(t, L)] = lax.shift_right_logical(
            dst_idx[j, pl.ds(t, L)], 14)
        return carry
    lax.fori_loop(0, DCH * (C // L), _unpack, 0)

    one16 = jnp.ones((L,), jnp.float32)
    z16 = jnp.zeros((L,), jnp.float32)

    def _fill(i, carry):
        for jj in range(H // L):
            ones_b[i, pl.ds(jj * L, L)] = one16
            zero_b[i, pl.ds(jj * L, L)] = z16
        return carry
    lax.fori_loop(0, C, _fill, 0)
    for k in range(RPT // C):
        pltpu.sync_copy(zero_b, deg_acc.at[pl.ds(s * RPT + k * C, C)])

    plsc.subcore_barrier()

    def _step(j, carry):
        pltpu.sync_copy(ones_b, deg_acc.at[dst_idx.at[j]], add=True)
        return carry
    lax.fori_loop(0, DCH, _step, 0)

    plsc.subcore_barrier()

    pltpu.sync_copy(deg_acc.at[pl.ds(s * RPT, RPT)],
                    deg_out.at[c, pl.ds(s * RPT, RPT)])


_deg = pl.kernel(
    _deg_body,
    out_type=jax.ShapeDtypeStruct((NC, NP, H), jnp.float32),
    mesh=_MESH,
    scratch_types=[
        pltpu.VMEM((DCH, C), jnp.int32),
        pltpu.VMEM((C, H), jnp.float32),
        pltpu.VMEM((C, H), jnp.float32),
        pltpu.VMEM_SHARED((NP, H), jnp.float32),
    ],
)


def _encoder_body(x_ref, w0_ref, b0_ref, w1_ref, b1_ref, out_ref):
    h = jnp.maximum(
        jnp.dot(x_ref[...], w0_ref[...], preferred_element_type=jnp.float32)
        + b0_ref[...], 0.0)
    out_ref[...] = jnp.maximum(
        jnp.dot(h, w1_ref[...], preferred_element_type=jnp.float32)
        + b1_ref[...], 0.0)


def _encoder(x, w0, b0, w1, b1):
    return pl.pallas_call(
        _encoder_body,
        out_shape=jax.ShapeDtypeStruct((N, H), jnp.float32),
    )(x, w0, b0.reshape(1, H), w1, b1.reshape(1, H))


def _mean_msg(m_ref, degp_ref):
    dp = degp_ref[...]
    deg = dp[0, :N, 0] + dp[1, :N, 0]
    dinv = 1.0 / jnp.maximum(deg, 1.0)
    return (m_ref[0, :N, :] + m_ref[1, :N, :]) * dinv[:, None]


def _conv_body(m_ref, degp_ref, h_ref, w_ref, b_ref, out_ref):
    m = _mean_msg(m_ref, degp_ref)
    out_ref[...] = jnp.maximum(
        jnp.dot(m, w_ref[...], preferred_element_type=jnp.float32)
        + b_ref[...] + h_ref[...], 0.0)


def _conv(m, degp, h, w, b):
    return pl.pallas_call(
        _conv_body,
        out_shape=jax.ShapeDtypeStruct((N, H), jnp.float32),
    )(m, degp, h, w, b.reshape(1, H))


def _conv_pred_body(m_ref, degp_ref, h_ref, w_ref, b_ref,
                    pw0_ref, pb0_ref, pw1_ref, pb1_ref, out_ref):
    m = _mean_msg(m_ref, degp_ref)
    hn = jnp.maximum(
        jnp.dot(m, w_ref[...], preferred_element_type=jnp.float32)
        + b_ref[...] + h_ref[...], 0.0)
    obj = jnp.mean(hn, axis=0, keepdims=True)
    z = jnp.maximum(
        jnp.dot(obj, pw0_ref[...], preferred_element_type=jnp.float32)
        + pb0_ref[...], 0.0)
    out_ref[...] = (
        jnp.dot(z, pw1_ref[...], preferred_element_type=jnp.float32)
        + pb1_ref[...])


def _conv_pred(m, degp, h, w, b, pw0, pb0, pw1, pb1):
    out = pl.pallas_call(
        _conv_pred_body,
        out_shape=jax.ShapeDtypeStruct((1, 1), jnp.float32),
    )(m, degp, h, w, b.reshape(1, H),
      pw0, pb0.reshape(1, H), pw1, pb1.reshape(1, 1))
    return out.reshape(())


def kernel(x, edge_index, enc_W0, enc_b0, enc_W1, enc_b1,
           conv_W0, conv_b0, conv_W1, conv_b1, conv_W2, conv_b2,
           pred_W0, pred_b0, pred_W1, pred_b1):
    src = edge_index[0]
    dst = edge_index[1]
    pad = EPAD - E
    src_p = jnp.concatenate([src, jnp.arange(pad, dtype=jnp.int32) % N])
    dst_p = jnp.concatenate(
        [dst, N + (jnp.arange(pad, dtype=jnp.int32) % (NP - N))])
    ei_p = (src_p | (dst_p << 14)).reshape(NCH, C)

    h = _encoder(x, enc_W0, enc_b0, enc_W1, enc_b1)
    degp = _deg(ei_p)

    m = _msg(h, ei_p)
    h = _conv(m, degp, h, conv_W0, conv_b0)
    m = _msg(h, ei_p)
    h = _conv(m, degp, h, conv_W1, conv_b1)
    m = _msg(h, ei_p)
    return _conv_pred(m, degp, h, conv_W2, conv_b2,
                      pred_W0, pred_b0, pred_W1, pred_b1)
